# baseline (speedup 1.0000x reference)
import jax
import jax.numpy as jnp
from jax.experimental import pallas as pl
from jax.experimental.pallas import tpu as pltpu


def _max_c_kernel(x_ref, w2_ref, b_ref, c_ref, acc_ref):
    i = pl.program_id(0)
    t = jnp.max(x_ref[...], axis=0, keepdims=True)

    @pl.when(i == 0)
    def _():
        acc_ref[...] = t

    @pl.when(i > 0)
    def _():
        acc_ref[...] = jnp.maximum(acc_ref[...], t)

    @pl.when(i == pl.num_programs(0) - 1)
    def _():
        c_ref[...] = (
            jnp.dot(acc_ref[...], w2_ref[...], preferred_element_type=jnp.float32)
            + b_ref[...]
        )


def _mm_kernel(x_ref, w1_ref, c_ref, o_ref):
    o_ref[...] = jnp.maximum(
        jnp.dot(x_ref[...], w1_ref[...], preferred_element_type=jnp.float32)
        + c_ref[...],
        0.0,
    )


def _pick_tile(e, target):
    t = min(target, e)
    while e % t:
        t -= 8
    return t


def kernel(edge_pred, edge_corner, all_corners, edge_x, image_x, W, b):
    E, D = edge_x.shape
    W1 = W[:D]
    W2 = W[D:]
    b2 = b.reshape(1, D)

    t1 = _pick_tile(E, 2000)
    c = pl.pallas_call(
        _max_c_kernel,
        grid=(E // t1,),
        in_specs=[
            pl.BlockSpec((t1, D), lambda i: (i, 0)),
            pl.BlockSpec((D, D), lambda i: (0, 0)),
            pl.BlockSpec((1, D), lambda i: (0, 0)),
        ],
        out_specs=pl.BlockSpec((1, D), lambda i: (0, 0)),
        out_shape=jax.ShapeDtypeStruct((1, D), jnp.float32),
        scratch_shapes=[pltpu.VMEM((1, D), jnp.float32)],
    )(edge_x, W2, b2)

    t2 = _pick_tile(E, 2000)
    out = pl.pallas_call(
        _mm_kernel,
        grid=(E // t2,),
        in_specs=[
            pl.BlockSpec((t2, D), lambda i: (i, 0)),
            pl.BlockSpec((D, D), lambda i: (0, 0)),
            pl.BlockSpec((1, D), lambda i: (0, 0)),
        ],
        out_specs=pl.BlockSpec((t2, D), lambda i: (i, 0)),
        out_shape=jax.ShapeDtypeStruct((E, D), jnp.float32),
    )(edge_x, W1, c)
    return out
